# baseline (speedup 1.0000x reference)
import functools

import jax
import jax.numpy as jnp
from jax import lax
from jax.experimental import pallas as pl
from jax.experimental.pallas import tpu as pltpu
from jax.experimental.pallas import tpu_sc as plsc

EMBED = 128
SEQ = 200
NW = 32
CH = 32
NBUF = 8


def _prep_body(xn_ref, xp_ref, xpb_ref, xrt_ref, wcat_ref, cidx_ref, tf_ref):
    j = lax.broadcasted_iota(jnp.int32, (136, EMBED), 0)
    k = lax.broadcasted_iota(jnp.int32, (136, EMBED), 1)
    g = ((k == j // 27).astype(jnp.float32)
         + (k == 5 + (j // 9) % 3).astype(jnp.float32)
         + (k == 8 + (j // 3) % 3).astype(jnp.float32)
         + (k == 11 + j % 3).astype(jnp.float32))
    g = jnp.where(j < 135, g, 0.0)
    tf_ref[...] = jnp.dot(g, wcat_ref[...], preferred_element_type=jnp.float32)
    cidx_ref[...] = ((xn_ref[...] * 3 + xp_ref[...]) * 3
                     + xpb_ref[...]) * 3 + xrt_ref[...]


def _prep(xn, xp, xpb, xrt, wcat):
    b = xn.shape[0]
    bs = min(b, 1024)
    idx_spec = pl.BlockSpec((bs, SEQ), lambda i: (i, 0))
    return pl.pallas_call(
        _prep_body,
        grid=(b // bs,),
        in_specs=[idx_spec, idx_spec, idx_spec, idx_spec,
                  pl.BlockSpec((EMBED, EMBED), lambda i: (0, 0))],
        out_specs=[pl.BlockSpec((bs, SEQ), lambda i: (i, 0)),
                   pl.BlockSpec((136, EMBED), lambda i: (0, 0))],
        out_shape=[jax.ShapeDtypeStruct((b, SEQ), jnp.int32),
                   jax.ShapeDtypeStruct((136, EMBED), jnp.float32)],
    )(xn, xp, xpb, xrt, wcat)


def _sc_body(tf_hbm, idx_hbm, out_hbm, tf_v, idx_v, *rest):
    bufs = rest[:NBUF]
    gsems = rest[NBUF:2 * NBUF]
    ssems = rest[2 * NBUF:3 * NBUF]
    nrows = idx_hbm.shape[0] // NW
    nch = nrows // CH
    sid = lax.axis_index("s")
    wid = sid * 2 + lax.axis_index("c")
    base = wid * nrows

    @pl.when(sid == 0)
    def _():
        pltpu.sync_copy(tf_hbm, tf_v)
    plsc.subcore_barrier()
    pltpu.sync_copy(idx_hbm.at[pl.ds(base, nrows)], idx_v)

    def start_gather(j, b):
        sl = idx_v.at[pl.ds(j * CH, CH)]
        pltpu.async_copy(tf_v.at[sl], bufs[b], gsems[b])

    for b in range(NBUF):
        start_gather(b, b)

    def rnd(t, carry):
        for b in range(NBUF):
            j = NBUF * t + b
            pltpu.make_async_copy(tf_v.at[idx_v.at[pl.ds(0, CH)]],
                                  bufs[b], gsems[b]).wait()
            pltpu.async_copy(bufs[b],
                             out_hbm.at[pl.ds(base + j * CH, CH)], ssems[b])

            @pl.when(j + NBUF < nch)
            def _():
                pltpu.make_async_copy(
                    bufs[b], out_hbm.at[pl.ds(0, CH)], ssems[b]).wait()
                start_gather(j + NBUF, b)
        return carry

    lax.fori_loop(0, nch // NBUF, rnd, 0)
    for b in range(NBUF):
        pltpu.make_async_copy(bufs[b], out_hbm.at[pl.ds(0, CH)],
                              ssems[b]).wait()


def _sc_expand(tf, cidx_flat):
    n = cidx_flat.shape[0]
    nrows = n // NW
    mesh = plsc.VectorSubcoreMesh(core_axis_name="c", subcore_axis_name="s")
    f = functools.partial(
        pl.kernel,
        mesh=mesh,
        out_type=jax.ShapeDtypeStruct((n, EMBED), jnp.float32),
        scratch_types=(
            [pltpu.VMEM_SHARED((136, EMBED), jnp.float32),
             pltpu.VMEM((nrows,), jnp.int32)]
            + [pltpu.VMEM((CH, EMBED), jnp.float32)] * NBUF
            + [pltpu.SemaphoreType.DMA] * (2 * NBUF)
        ),
    )(_sc_body)
    return f(tf, cidx_flat)


@jax.jit
def kernel(X_nucl, X_proto, X_pbs, X_rt, We, Wproto, Wpbs, Wrt):
    B, L = X_nucl.shape
    xn = X_nucl.astype(jnp.int32)
    xp = X_proto.astype(jnp.int32)
    xpb = X_pbs.astype(jnp.int32)
    xrt = X_rt.astype(jnp.int32)
    wcat = jnp.zeros((EMBED, EMBED), jnp.float32)
    wcat = wcat.at[0:5].set(We).at[5:8].set(Wproto)
    wcat = wcat.at[8:11].set(Wpbs).at[11:14].set(Wrt)

    cidx, tf = _prep(xn, xp, xpb, xrt, wcat)
    out_sc = _sc_expand(tf, cidx.reshape(B * SEQ))
    return out_sc.reshape(B, SEQ, EMBED)


# speedup vs baseline: 1.1065x; 1.1065x over previous
import functools

import jax
import jax.numpy as jnp
from jax import lax
from jax.experimental import pallas as pl
from jax.experimental.pallas import tpu as pltpu
from jax.experimental.pallas import tpu_sc as plsc

EMBED = 128
SEQ = 200
NW = 32
CH = 64
NBUF = 8


def _prep_body(xn_ref, xp_ref, xpb_ref, xrt_ref, wcat_ref, cidx_ref, tf_ref):
    j = lax.broadcasted_iota(jnp.int32, (136, EMBED), 0)
    k = lax.broadcasted_iota(jnp.int32, (136, EMBED), 1)
    g = ((k == j // 27).astype(jnp.float32)
         + (k == 5 + (j // 9) % 3).astype(jnp.float32)
         + (k == 8 + (j // 3) % 3).astype(jnp.float32)
         + (k == 11 + j % 3).astype(jnp.float32))
    g = jnp.where(j < 135, g, 0.0)
    tf_ref[...] = jnp.dot(g, wcat_ref[...], preferred_element_type=jnp.float32)
    cidx_ref[...] = ((xn_ref[...] * 3 + xp_ref[...]) * 3
                     + xpb_ref[...]) * 3 + xrt_ref[...]


def _prep(xn, xp, xpb, xrt, wcat):
    b = xn.shape[0]
    bs = min(b, 1024)
    idx_spec = pl.BlockSpec((bs, SEQ), lambda i: (i, 0))
    return pl.pallas_call(
        _prep_body,
        grid=(b // bs,),
        in_specs=[idx_spec, idx_spec, idx_spec, idx_spec,
                  pl.BlockSpec((EMBED, EMBED), lambda i: (0, 0))],
        out_specs=[pl.BlockSpec((bs, SEQ), lambda i: (i, 0)),
                   pl.BlockSpec((136, EMBED), lambda i: (0, 0))],
        out_shape=[jax.ShapeDtypeStruct((b, SEQ), jnp.int32),
                   jax.ShapeDtypeStruct((136, EMBED), jnp.float32)],
    )(xn, xp, xpb, xrt, wcat)


def _sc_body(tf_hbm, idx_hbm, out_hbm, tf_v, idx_v, *rest):
    bufs = rest[:NBUF]
    gsems = rest[NBUF:2 * NBUF]
    ssems = rest[2 * NBUF:3 * NBUF]
    nrows = idx_hbm.shape[0] // NW
    nch = nrows // CH
    sid = lax.axis_index("s")
    wid = sid * 2 + lax.axis_index("c")
    base = wid * nrows

    @pl.when(sid == 0)
    def _():
        pltpu.sync_copy(tf_hbm, tf_v)
    plsc.subcore_barrier()
    pltpu.sync_copy(idx_hbm.at[pl.ds(base, nrows)], idx_v)

    def start_gather(j, b):
        sl = idx_v.at[pl.ds(j * CH, CH)]
        pltpu.async_copy(tf_v.at[sl], bufs[b], gsems[b])

    for b in range(NBUF):
        start_gather(b, b)

    def rnd(t, carry):
        for b in range(NBUF):
            j = NBUF * t + b
            pltpu.make_async_copy(tf_v.at[idx_v.at[pl.ds(0, CH)]],
                                  bufs[b], gsems[b]).wait()
            pltpu.async_copy(bufs[b],
                             out_hbm.at[pl.ds(base + j * CH, CH)], ssems[b])

            @pl.when(j + NBUF < nch)
            def _():
                pltpu.make_async_copy(
                    bufs[b], out_hbm.at[pl.ds(0, CH)], ssems[b]).wait()
                start_gather(j + NBUF, b)
        return carry

    lax.fori_loop(0, nch // NBUF, rnd, 0)
    for b in range(NBUF):
        pltpu.make_async_copy(bufs[b], out_hbm.at[pl.ds(0, CH)],
                              ssems[b]).wait()


def _sc_expand(tf, cidx_flat):
    n = cidx_flat.shape[0]
    nrows = n // NW
    mesh = plsc.VectorSubcoreMesh(core_axis_name="c", subcore_axis_name="s")
    f = functools.partial(
        pl.kernel,
        mesh=mesh,
        out_type=jax.ShapeDtypeStruct((n, EMBED), jnp.float32),
        scratch_types=(
            [pltpu.VMEM_SHARED((136, EMBED), jnp.float32),
             pltpu.VMEM((nrows,), jnp.int32)]
            + [pltpu.VMEM((CH, EMBED), jnp.float32)] * NBUF
            + [pltpu.SemaphoreType.DMA] * (2 * NBUF)
        ),
    )(_sc_body)
    return f(tf, cidx_flat)


@jax.jit
def kernel(X_nucl, X_proto, X_pbs, X_rt, We, Wproto, Wpbs, Wrt):
    B, L = X_nucl.shape
    xn = X_nucl.astype(jnp.int32)
    xp = X_proto.astype(jnp.int32)
    xpb = X_pbs.astype(jnp.int32)
    xrt = X_rt.astype(jnp.int32)
    wcat = jnp.zeros((EMBED, EMBED), jnp.float32)
    wcat = wcat.at[0:5].set(We).at[5:8].set(Wproto)
    wcat = wcat.at[8:11].set(Wpbs).at[11:14].set(Wrt)

    cidx, tf = _prep(xn, xp, xpb, xrt, wcat)
    out_sc = _sc_expand(tf, cidx.reshape(B * SEQ))
    return out_sc.reshape(B, SEQ, EMBED)
